# Initial kernel scaffold; baseline (speedup 1.0000x reference)
import functools
import jax
import jax.numpy as jnp
from jax import lax
from jax.experimental import pallas as pl
from jax.experimental.pallas import tpu as pltpu
from jax.experimental.pallas import tpu_sc as plsc

F32 = jnp.float32

NC = 2
NS = 16
NW = NC * NS
L = 16

_MESH = plsc.VectorSubcoreMesh(core_axis_name="c", subcore_axis_name="s")


def _dense_body(h_ref, Wa_ref, ba_ref, ws_ref, Wv_ref, bv_ref, s_ref, x_ref):
    hb = h_ref[...]
    e = jnp.tanh(jnp.dot(hb, Wa_ref[...], preferred_element_type=F32)
                 + ba_ref[...])
    s_ref[...] = jnp.sum(e * ws_ref[...], axis=1)
    x_ref[...] = jnp.tanh(jnp.dot(hb, Wv_ref[...], preferred_element_type=F32)
                          + bv_ref[...])


def _norm_rows(o, d):
    mu = jnp.mean(o, axis=1, keepdims=True)
    var = jnp.sum((o - mu) * (o - mu), axis=1, keepdims=True) * (1.0 / (d - 1))
    return o / jnp.sqrt(var)


def _norm_dense_body(p_ref, Wa_ref, ba_ref, ws_ref, Wv_ref, bv_ref,
                     s_ref, x_ref):
    o = p_ref[0] + p_ref[1]
    hb = _norm_rows(o, o.shape[1])
    e = jnp.tanh(jnp.dot(hb, Wa_ref[...], preferred_element_type=F32)
                 + ba_ref[...])
    s_ref[...] = jnp.sum(e * ws_ref[...], axis=1)
    x_ref[...] = jnp.tanh(jnp.dot(hb, Wv_ref[...], preferred_element_type=F32)
                          + bv_ref[...])


def _final_body(p_ref, out_ref):
    o = p_ref[0] + p_ref[1]
    out_ref[...] = _norm_rows(o, o.shape[1])


def _rsum_body(p_ref, rinv_ref):
    rinv_ref[...] = 1.0 / jnp.sum(p_ref[...], axis=0)


def _tc_dense(h, Wa, ba, ws, Wv, bv, blk):
    n, d = h.shape
    grid = (n // blk,)
    wspec = pl.BlockSpec((d, d), lambda i: (0, 0))
    bspec = pl.BlockSpec((1, d), lambda i: (0, 0))
    return pl.pallas_call(
        _dense_body,
        grid=grid,
        in_specs=[pl.BlockSpec((blk, d), lambda i: (i, 0)),
                  wspec, bspec, bspec, wspec, bspec],
        out_specs=[pl.BlockSpec((blk,), lambda i: (i,)),
                   pl.BlockSpec((blk, d), lambda i: (i, 0))],
        out_shape=[jax.ShapeDtypeStruct((n,), F32),
                   jax.ShapeDtypeStruct((n, d), F32)],
    )(h, Wa, ba, ws, Wv, bv)


def _tc_norm_dense(p, Wa, ba, ws, Wv, bv, blk):
    _, n, d = p.shape
    grid = (n // blk,)
    wspec = pl.BlockSpec((d, d), lambda i: (0, 0))
    bspec = pl.BlockSpec((1, d), lambda i: (0, 0))
    return pl.pallas_call(
        _norm_dense_body,
        grid=grid,
        in_specs=[pl.BlockSpec((2, blk, d), lambda i: (0, i, 0)),
                  wspec, bspec, bspec, wspec, bspec],
        out_specs=[pl.BlockSpec((blk,), lambda i: (i,)),
                   pl.BlockSpec((blk, d), lambda i: (i, 0))],
        out_shape=[jax.ShapeDtypeStruct((n,), F32),
                   jax.ShapeDtypeStruct((n, d), F32)],
    )(p, Wa, ba, ws, Wv, bv)


def _tc_final(p, blk):
    _, n, d = p.shape
    grid = (n // blk,)
    return pl.pallas_call(
        _final_body,
        grid=grid,
        in_specs=[pl.BlockSpec((2, blk, d), lambda i: (0, i, 0))],
        out_specs=pl.BlockSpec((blk, d), lambda i: (i, 0)),
        out_shape=jax.ShapeDtypeStruct((n, d), F32),
    )(p)


def _tc_rsum(parts):
    nw, n = parts.shape
    return pl.pallas_call(
        _rsum_body,
        in_specs=[pl.BlockSpec((nw, n), lambda: (0, 0))],
        out_specs=pl.BlockSpec((n,), lambda: (0,)),
        out_shape=jax.ShapeDtypeStruct((n,), F32),
    )(parts)


def _make_edge_a(n, e):
    ept = e // NW

    @functools.partial(
        pl.kernel,
        mesh=_MESH,
        out_type=(jax.ShapeDtypeStruct((e,), F32),
                  jax.ShapeDtypeStruct((NW, n), F32)),
        scratch_types=[pltpu.VMEM((n,), F32),
                       pltpu.VMEM((n,), F32),
                       pltpu.VMEM((ept,), jnp.int32),
                       pltpu.VMEM((ept,), jnp.int32),
                       pltpu.VMEM((ept,), F32)],
    )
    def edge_a(s_hbm, row_hbm, col_hbm, ex_hbm, rpart_hbm,
               s_v, r_v, row_v, col_v, ex_v):
        cid = lax.axis_index("c")
        sid = lax.axis_index("s")
        wid = sid * NC + cid
        base = wid * ept
        pltpu.sync_copy(s_hbm, s_v)
        pltpu.sync_copy(row_hbm.at[pl.ds(base, ept)], row_v)
        pltpu.sync_copy(col_hbm.at[pl.ds(base, ept)], col_v)

        zero16 = jnp.zeros((L,), F32)

        def zb(i, carry):
            r_v[pl.ds(i * L, L)] = zero16
            return carry
        lax.fori_loop(0, n // L, zb, 0)

        def eb(j, carry):
            off = j * L
            r16 = row_v[pl.ds(off, L)]
            c16 = col_v[pl.ds(off, L)]
            sr = plsc.load_gather(s_v, [r16])
            sc_ = plsc.load_gather(s_v, [c16])
            z = sr + sc_
            u = jnp.exp(z + z)
            t = 1.0 - 2.0 / (u + 1.0)
            ex = jnp.exp(t)
            ex_v[pl.ds(off, L)] = ex
            plsc.addupdate_scatter(r_v, [r16], ex)
            return carry
        lax.fori_loop(0, ept // L, eb, 0)

        pltpu.sync_copy(ex_v, ex_hbm.at[pl.ds(base, ept)])
        pltpu.sync_copy(r_v, rpart_hbm.at[wid])

    return edge_a


def _make_edge_b(n, e, d):
    ept = e // NW
    cg = 80
    nch = ept // cg
    rpt = n // NS
    zrows = 125
    nz = rpt // zrows

    @functools.partial(
        pl.kernel,
        mesh=_MESH,
        out_type=jax.ShapeDtypeStruct((NC, n, d), F32),
        scratch_types=[pltpu.VMEM((n,), F32),
                       pltpu.VMEM((nch, cg), jnp.int32),
                       pltpu.VMEM((nch, cg), jnp.int32),
                       pltpu.VMEM((nch, cg), F32),
                       pltpu.VMEM((cg, d), F32),
                       pltpu.VMEM((cg,), F32),
                       pltpu.VMEM((zrows, d), F32),
                       pltpu.VMEM_SHARED((n, d), F32),
                       pltpu.SemaphoreType.DMA],
    )
    def edge_b(x_hbm, row_hbm, col_hbm, ex_hbm, rinv_hbm, opart_hbm,
               rinv_v, row_v, col_v, ex_v, xbuf, wbuf, zbuf, acc, sem):
        cid = lax.axis_index("c")
        sid = lax.axis_index("s")
        wid = sid * NC + cid

        pltpu.sync_copy(rinv_hbm, rinv_v)
        pltpu.sync_copy(row_hbm.at[wid], row_v)
        pltpu.sync_copy(col_hbm.at[wid], col_v)
        pltpu.sync_copy(ex_hbm.at[wid], ex_v)

        zero16 = jnp.zeros((L,), F32)
        dl = d // L

        def zb(i, carry):
            zbuf[i // dl, pl.ds((i % dl) * L, L)] = zero16
            return carry
        lax.fori_loop(0, zrows * dl, zb, 0)

        def zc(k, carry):
            pltpu.sync_copy(zbuf, acc.at[pl.ds(sid * rpt + k * zrows, zrows)])
            return carry
        lax.fori_loop(0, nz, zc, 0)
        plsc.subcore_barrier()

        def chunk(c, carry):
            def wb(j, carry2):
                r16 = row_v[c, pl.ds(j * L, L)]
                e16 = ex_v[c, pl.ds(j * L, L)]
                ri = plsc.load_gather(rinv_v, [r16])
                wbuf[pl.ds(j * L, L)] = e16 * ri
                return carry2
            lax.fori_loop(0, cg // L, wb, 0)

            pltpu.async_copy(x_hbm.at[col_v.at[c]], xbuf, sem).wait()

            def rb(i, carry2):
                w = wbuf[i]
                for j in range(dl):
                    xbuf[i, pl.ds(j * L, L)] = xbuf[i, pl.ds(j * L, L)] * w
                return carry2
            lax.fori_loop(0, cg, rb, 0)

            pltpu.sync_copy(xbuf, acc.at[row_v.at[c]], add=True)
            return carry
        lax.fori_loop(0, nch, chunk, 0)
        plsc.subcore_barrier()

        def ob(k, carry):
            off = sid * rpt + k * zrows
            pltpu.sync_copy(acc.at[pl.ds(off, zrows)], zbuf)
            pltpu.sync_copy(zbuf, opart_hbm.at[cid, pl.ds(off, zrows)])
            return carry
        lax.fori_loop(0, nz, ob, 0)

    return edge_b


def kernel(h, edge_index, W11, b11, W12, b12, W13, b13,
           W21, b21, W22, b22, W23, b23):
    n, d = h.shape
    e = edge_index.shape[1]
    blk = 2000

    row = edge_index[0]
    col = edge_index[1]
    ept = e // NW
    cg = 80
    nch = ept // cg
    row3 = row.reshape(NW, nch, cg)
    col3 = col.reshape(NW, nch, cg)

    edge_a = _make_edge_a(n, e)
    edge_b = _make_edge_b(n, e, d)

    def layer(s, x):
        ex, rparts = edge_a(s, row, col)
        rinv = _tc_rsum(rparts)
        ex3 = ex.reshape(NW, nch, cg)
        return edge_b(x, row3, col3, ex3, rinv)

    s1, x1 = _tc_dense(h, W11, b11.reshape(1, d), W12.reshape(1, d),
                       W13, b13.reshape(1, d), blk)
    s1 = s1 + 0.5 * b12[0]
    opart1 = layer(s1, x1)

    s2, x2 = _tc_norm_dense(opart1, W21, b21.reshape(1, d),
                            W22.reshape(1, d), W23, b23.reshape(1, d), blk)
    s2 = s2 + 0.5 * b22[0]
    opart2 = layer(s2, x2)

    return _tc_final(opart2, blk)


# speedup vs baseline: 14.3639x; 14.3639x over previous
import functools
import jax
import jax.numpy as jnp
from jax import lax
from jax.experimental import pallas as pl
from jax.experimental.pallas import tpu as pltpu
from jax.experimental.pallas import tpu_sc as plsc

F32 = jnp.float32

NC = 2
NS = 16
NW = NC * NS
L = 16

_MESH = plsc.VectorSubcoreMesh(core_axis_name="c", subcore_axis_name="s")


def _dense_body(h_ref, Wa_ref, ba_ref, ws_ref, Wv_ref, bv_ref, s_ref, x_ref):
    hb = h_ref[...]
    e = jnp.tanh(jnp.dot(hb, Wa_ref[...], preferred_element_type=F32)
                 + ba_ref[...])
    s_ref[...] = jnp.sum(e * ws_ref[...], axis=1)[:, None]
    x_ref[...] = jnp.tanh(jnp.dot(hb, Wv_ref[...], preferred_element_type=F32)
                          + bv_ref[...])


def _norm_rows(o, d):
    mu = jnp.mean(o, axis=1, keepdims=True)
    var = jnp.sum((o - mu) * (o - mu), axis=1, keepdims=True) * (1.0 / (d - 1))
    return o / jnp.sqrt(var)


def _norm_dense_body(p_ref, Wa_ref, ba_ref, ws_ref, Wv_ref, bv_ref,
                     s_ref, x_ref):
    o = p_ref[0] + p_ref[1]
    hb = _norm_rows(o, o.shape[1])
    e = jnp.tanh(jnp.dot(hb, Wa_ref[...], preferred_element_type=F32)
                 + ba_ref[...])
    s_ref[...] = jnp.sum(e * ws_ref[...], axis=1)[:, None]
    x_ref[...] = jnp.tanh(jnp.dot(hb, Wv_ref[...], preferred_element_type=F32)
                          + bv_ref[...])


def _final_body(p_ref, out_ref):
    o = p_ref[0] + p_ref[1]
    out_ref[...] = _norm_rows(o, o.shape[1])


def _rsum_body(p_ref, rinv_ref):
    rinv_ref[...] = 1.0 / jnp.sum(p_ref[...], axis=0)


def _tc_dense(h, Wa, ba, ws, Wv, bv, blk):
    n, d = h.shape
    grid = (n // blk,)
    wspec = pl.BlockSpec((d, d), lambda i: (0, 0))
    bspec = pl.BlockSpec((1, d), lambda i: (0, 0))
    return pl.pallas_call(
        _dense_body,
        grid=grid,
        in_specs=[pl.BlockSpec((blk, d), lambda i: (i, 0)),
                  wspec, bspec, bspec, wspec, bspec],
        out_specs=[pl.BlockSpec((blk, 1), lambda i: (i, 0)),
                   pl.BlockSpec((blk, d), lambda i: (i, 0))],
        out_shape=[jax.ShapeDtypeStruct((n, 1), F32),
                   jax.ShapeDtypeStruct((n, d), F32)],
    )(h, Wa, ba, ws, Wv, bv)


def _tc_norm_dense(p, Wa, ba, ws, Wv, bv, blk, n):
    d = p.shape[2]
    grid = (n // blk,)
    wspec = pl.BlockSpec((d, d), lambda i: (0, 0))
    bspec = pl.BlockSpec((1, d), lambda i: (0, 0))
    return pl.pallas_call(
        _norm_dense_body,
        grid=grid,
        in_specs=[pl.BlockSpec((NC, blk, d), lambda i: (0, i, 0)),
                  wspec, bspec, bspec, wspec, bspec],
        out_specs=[pl.BlockSpec((blk, 1), lambda i: (i, 0)),
                   pl.BlockSpec((blk, d), lambda i: (i, 0))],
        out_shape=[jax.ShapeDtypeStruct((n, 1), F32),
                   jax.ShapeDtypeStruct((n, d), F32)],
    )(p, Wa, ba, ws, Wv, bv)


def _tc_final(p, blk, n):
    d = p.shape[2]
    grid = (n // blk,)
    return pl.pallas_call(
        _final_body,
        grid=grid,
        in_specs=[pl.BlockSpec((NC, blk, d), lambda i: (0, i, 0))],
        out_specs=pl.BlockSpec((blk, d), lambda i: (i, 0)),
        out_shape=jax.ShapeDtypeStruct((n, d), F32),
    )(p)


def _tc_rsum(parts):
    nw, n = parts.shape
    return pl.pallas_call(
        _rsum_body,
        in_specs=[pl.BlockSpec((nw, n), lambda: (0, 0))],
        out_specs=pl.BlockSpec((n,), lambda: (0,)),
        out_shape=jax.ShapeDtypeStruct((n,), F32),
    )(parts)


def _make_edge_a(n, e):
    ept = e // NW

    @functools.partial(
        pl.kernel,
        mesh=_MESH,
        out_type=(jax.ShapeDtypeStruct((e,), F32),
                  jax.ShapeDtypeStruct((NW, n), F32)),
        scratch_types=[pltpu.VMEM((n,), F32),
                       pltpu.VMEM((n,), F32),
                       pltpu.VMEM((ept,), jnp.int32),
                       pltpu.VMEM((ept,), jnp.int32),
                       pltpu.VMEM((ept,), F32)],
        compiler_params=pltpu.CompilerParams(needs_layout_passes=False),
    )
    def edge_a(s_hbm, row_hbm, col_hbm, ex_hbm, rpart_hbm,
               s_v, r_v, row_v, col_v, ex_v):
        cid = lax.axis_index("c")
        sid = lax.axis_index("s")
        wid = sid * NC + cid
        base = wid * ept
        pltpu.sync_copy(s_hbm, s_v)
        pltpu.sync_copy(row_hbm.at[pl.ds(base, ept)], row_v)
        pltpu.sync_copy(col_hbm.at[pl.ds(base, ept)], col_v)

        zero16 = jnp.zeros((L,), F32)

        def zb(i, carry):
            r_v[pl.ds(i * L, L)] = zero16
            return carry
        lax.fori_loop(0, n // L, zb, 0)

        def eb(j, carry):
            off = j * L
            r16 = row_v[pl.ds(off, L)]
            c16 = col_v[pl.ds(off, L)]
            sr = plsc.load_gather(s_v, [r16])
            sc_ = plsc.load_gather(s_v, [c16])
            z = sr + sc_
            u = jnp.exp(z + z)
            t = 1.0 - 2.0 / (u + 1.0)
            ex = jnp.exp(t)
            ex_v[pl.ds(off, L)] = ex
            plsc.addupdate_scatter(r_v, [r16], ex)
            return carry
        lax.fori_loop(0, ept // L, eb, 0)

        pltpu.sync_copy(ex_v, ex_hbm.at[pl.ds(base, ept)])
        pltpu.sync_copy(r_v, rpart_hbm.at[wid])

    return edge_a


def _make_edge_w(n, e):
    ept = e // NW
    cw = 2000
    ncw = ept // cw

    @functools.partial(
        pl.kernel,
        mesh=_MESH,
        out_type=jax.ShapeDtypeStruct((e,), F32),
        scratch_types=[pltpu.VMEM((n,), F32),
                       pltpu.VMEM((cw,), jnp.int32),
                       pltpu.VMEM((cw,), F32)],
        compiler_params=pltpu.CompilerParams(needs_layout_passes=False),
    )
    def edge_w(rinv_hbm, row_hbm, ex_hbm, w_hbm, rinv_v, rowc, exc):
        cid = lax.axis_index("c")
        sid = lax.axis_index("s")
        wid = sid * NC + cid
        pltpu.sync_copy(rinv_hbm, rinv_v)

        def ck(c, carry):
            base = wid * ept + c * cw
            pltpu.sync_copy(row_hbm.at[pl.ds(base, cw)], rowc)
            pltpu.sync_copy(ex_hbm.at[pl.ds(base, cw)], exc)

            def g(j, carry2):
                off = j * L
                r16 = rowc[pl.ds(off, L)]
                ri = plsc.load_gather(rinv_v, [r16])
                exc[pl.ds(off, L)] = exc[pl.ds(off, L)] * ri
                return carry2
            lax.fori_loop(0, cw // L, g, 0)

            pltpu.sync_copy(exc, w_hbm.at[pl.ds(base, cw)])
            return carry
        lax.fori_loop(0, ncw, ck, 0)

    return edge_w


def _make_edge_b(n, e, d):
    ept = e // NW
    cg = 80
    nch = ept // cg
    zrows = 40
    rpt = -(-(n // NS) // zrows) * zrows
    nz = rpt // zrows
    npad = NS * rpt

    @functools.partial(
        pl.kernel,
        mesh=_MESH,
        out_type=jax.ShapeDtypeStruct((NC, npad, d), F32),
        scratch_types=[pltpu.VMEM((cg,), jnp.int32),
                       pltpu.VMEM((cg,), jnp.int32),
                       pltpu.VMEM((cg,), F32),
                       pltpu.VMEM((cg, d), F32),
                       pltpu.VMEM((zrows, d), F32),
                       pltpu.VMEM_SHARED((npad, d), F32),
                       pltpu.SemaphoreType.DMA],
        compiler_params=pltpu.CompilerParams(needs_layout_passes=False),
    )
    def edge_b(x_hbm, row_hbm, col_hbm, w_hbm, opart_hbm,
               rowc, colc, wc, xbuf, zobuf, acc, sem):
        cid = lax.axis_index("c")
        sid = lax.axis_index("s")
        wid = sid * NC + cid

        zero16 = jnp.zeros((L,), F32)
        dl = d // L
        gpr = cg // L

        def zb(i, carry):
            zobuf[i // dl, pl.ds((i % dl) * L, L)] = zero16
            return carry
        lax.fori_loop(0, zrows * dl, zb, 0)

        def zc(k, carry):
            pltpu.sync_copy(zobuf, acc.at[pl.ds(sid * rpt + k * zrows, zrows)])
            return carry
        lax.fori_loop(0, nz, zc, 0)
        plsc.subcore_barrier()

        def chunk(c, carry):
            base = wid * ept + c * cg
            pltpu.sync_copy(row_hbm.at[pl.ds(base, cg)], rowc)
            pltpu.sync_copy(col_hbm.at[pl.ds(base, cg)], colc)
            pltpu.sync_copy(w_hbm.at[pl.ds(base, cg)], wc)
            pltpu.async_copy(x_hbm.at[colc], xbuf, sem).wait()

            def rb(g, carry2):
                w16 = wc[pl.ds(g * L, L)]
                for t in range(L):
                    i = g * L + t
                    w = w16[t]
                    for j in range(dl):
                        xbuf[i, pl.ds(j * L, L)] = xbuf[i, pl.ds(j * L, L)] * w
                return carry2
            lax.fori_loop(0, gpr, rb, 0)

            pltpu.sync_copy(xbuf, acc.at[rowc], add=True)
            return carry
        lax.fori_loop(0, nch, chunk, 0)
        plsc.subcore_barrier()

        def ob(k, carry):
            off = sid * rpt + k * zrows
            pltpu.sync_copy(acc.at[pl.ds(off, zrows)], zobuf)
            pltpu.sync_copy(zobuf, opart_hbm.at[cid, pl.ds(off, zrows)])
            return carry
        lax.fori_loop(0, nz, ob, 0)

    return edge_b


def kernel(h, edge_index, W11, b11, W12, b12, W13, b13,
           W21, b21, W22, b22, W23, b23):
    n, d = h.shape
    e = edge_index.shape[1]
    blk = 2000

    row = edge_index[0]
    col = edge_index[1]

    edge_a = _make_edge_a(n, e)
    edge_w = _make_edge_w(n, e)
    edge_b = _make_edge_b(n, e, d)

    def layer(s, x):
        ex, rparts = edge_a(s, row, col)
        rinv = _tc_rsum(rparts)
        w = edge_w(rinv, row, ex)
        return edge_b(x, row, col, w)

    s1, x1 = _tc_dense(h, W11, b11.reshape(1, d), W12.reshape(1, d),
                       W13, b13.reshape(1, d), blk)
    s1 = s1.reshape(n) + 0.5 * b12[0]
    opart1 = layer(s1, x1)

    s2, x2 = _tc_norm_dense(opart1, W21, b21.reshape(1, d),
                            W22.reshape(1, d), W23, b23.reshape(1, d), blk, n)
    s2 = s2.reshape(n) + 0.5 * b22[0]
    opart2 = layer(s2, x2)

    return _tc_final(opart2, blk, n)


# speedup vs baseline: 27.5159x; 1.9156x over previous
import functools
import jax
import jax.numpy as jnp
from jax import lax
from jax.experimental import pallas as pl
from jax.experimental.pallas import tpu as pltpu
from jax.experimental.pallas import tpu_sc as plsc

F32 = jnp.float32

NC = 2
NS = 16
NW = NC * NS
L = 16

_MESH = plsc.VectorSubcoreMesh(core_axis_name="c", subcore_axis_name="s")


def _dense_body(h_ref, Wa_ref, ba_ref, ws_ref, Wv_ref, bv_ref, s_ref, x_ref):
    hb = h_ref[...]
    e = jnp.tanh(jnp.dot(hb, Wa_ref[...], preferred_element_type=F32)
                 + ba_ref[...])
    s_ref[...] = jnp.sum(e * ws_ref[...], axis=1)[:, None]
    x_ref[...] = jnp.tanh(jnp.dot(hb, Wv_ref[...], preferred_element_type=F32)
                          + bv_ref[...])


def _norm_rows(o, d):
    mu = jnp.mean(o, axis=1, keepdims=True)
    var = jnp.sum((o - mu) * (o - mu), axis=1, keepdims=True) * (1.0 / (d - 1))
    return o / jnp.sqrt(var)


def _norm_dense_body(p_ref, Wa_ref, ba_ref, ws_ref, Wv_ref, bv_ref,
                     s_ref, x_ref):
    o = p_ref[0] + p_ref[1]
    hb = _norm_rows(o, o.shape[1])
    e = jnp.tanh(jnp.dot(hb, Wa_ref[...], preferred_element_type=F32)
                 + ba_ref[...])
    s_ref[...] = jnp.sum(e * ws_ref[...], axis=1)[:, None]
    x_ref[...] = jnp.tanh(jnp.dot(hb, Wv_ref[...], preferred_element_type=F32)
                          + bv_ref[...])


def _final_body(p_ref, out_ref):
    o = p_ref[0] + p_ref[1]
    out_ref[...] = _norm_rows(o, o.shape[1])


def _tc_dense(h, Wa, ba, ws, Wv, bv, blk):
    n, d = h.shape
    grid = (n // blk,)
    wspec = pl.BlockSpec((d, d), lambda i: (0, 0))
    bspec = pl.BlockSpec((1, d), lambda i: (0, 0))
    return pl.pallas_call(
        _dense_body,
        grid=grid,
        in_specs=[pl.BlockSpec((blk, d), lambda i: (i, 0)),
                  wspec, bspec, bspec, wspec, bspec],
        out_specs=[pl.BlockSpec((blk, 1), lambda i: (i, 0)),
                   pl.BlockSpec((blk, d), lambda i: (i, 0))],
        out_shape=[jax.ShapeDtypeStruct((n, 1), F32),
                   jax.ShapeDtypeStruct((n, d), F32)],
    )(h, Wa, ba, ws, Wv, bv)


def _tc_norm_dense(p, Wa, ba, ws, Wv, bv, blk, n):
    d = p.shape[2]
    grid = (n // blk,)
    wspec = pl.BlockSpec((d, d), lambda i: (0, 0))
    bspec = pl.BlockSpec((1, d), lambda i: (0, 0))
    return pl.pallas_call(
        _norm_dense_body,
        grid=grid,
        in_specs=[pl.BlockSpec((NC, blk, d), lambda i: (0, i, 0)),
                  wspec, bspec, bspec, wspec, bspec],
        out_specs=[pl.BlockSpec((blk, 1), lambda i: (i, 0)),
                   pl.BlockSpec((blk, d), lambda i: (i, 0))],
        out_shape=[jax.ShapeDtypeStruct((n, 1), F32),
                   jax.ShapeDtypeStruct((n, d), F32)],
    )(p, Wa, ba, ws, Wv, bv)


def _tc_final(p, blk, n):
    d = p.shape[2]
    grid = (n // blk,)
    return pl.pallas_call(
        _final_body,
        grid=grid,
        in_specs=[pl.BlockSpec((NC, blk, d), lambda i: (0, i, 0))],
        out_specs=pl.BlockSpec((blk, d), lambda i: (i, 0)),
        out_shape=jax.ShapeDtypeStruct((n, d), F32),
    )(p)


def _make_edge_a(n, e):
    ept = e // NW

    @functools.partial(
        pl.kernel,
        mesh=_MESH,
        out_type=jax.ShapeDtypeStruct((e,), F32),
        scratch_types=[pltpu.VMEM((n,), F32),
                       pltpu.VMEM((ept,), jnp.int32),
                       pltpu.VMEM((ept,), jnp.int32),
                       pltpu.VMEM((ept,), F32)],
        compiler_params=pltpu.CompilerParams(needs_layout_passes=False),
    )
    def edge_a(s_hbm, row_hbm, col_hbm, ex_hbm, s_v, row_v, col_v, ex_v):
        cid = lax.axis_index("c")
        sid = lax.axis_index("s")
        wid = sid * NC + cid
        base = wid * ept
        pltpu.sync_copy(s_hbm, s_v)
        pltpu.sync_copy(row_hbm.at[pl.ds(base, ept)], row_v)
        pltpu.sync_copy(col_hbm.at[pl.ds(base, ept)], col_v)

        def eb(j, carry):
            off = j * L
            r16 = row_v[pl.ds(off, L)]
            c16 = col_v[pl.ds(off, L)]
            sr = plsc.load_gather(s_v, [r16])
            sc_ = plsc.load_gather(s_v, [c16])
            z = sr + sc_
            u = jnp.exp(z + z)
            t = 1.0 - 2.0 / (u + 1.0)
            ex_v[pl.ds(off, L)] = jnp.exp(t)
            return carry
        lax.fori_loop(0, ept // L, eb, 0)

        pltpu.sync_copy(ex_v, ex_hbm.at[pl.ds(base, ept)])

    return edge_a


def _make_edge_b(n, e, d):
    ept = e // NW
    cg = 80
    nch = ept // cg
    scg = 2000
    cps = scg // cg
    zrows = 16
    rpt = -(-(n // NS) // zrows) * zrows
    nz = rpt // zrows
    npad = NS * rpt

    @functools.partial(
        pl.kernel,
        mesh=_MESH,
        out_type=jax.ShapeDtypeStruct((NC, npad, d), F32),
        scratch_types=[pltpu.VMEM((2, cg), jnp.int32),
                       pltpu.VMEM((2, cg), jnp.int32),
                       pltpu.VMEM((scg,), F32),
                       pltpu.VMEM((cg, d), F32),
                       pltpu.VMEM((cg, d), F32),
                       pltpu.VMEM((zrows, d), F32),
                       pltpu.VMEM_SHARED((npad, d), F32),
                       pltpu.SemaphoreType.DMA,
                       pltpu.SemaphoreType.DMA],
        compiler_params=pltpu.CompilerParams(needs_layout_passes=False),
    )
    def edge_b(x_hbm, rc_hbm, ex_hbm, opart_hbm,
               idx0, idx1, exb, xb0, xb1, zbuf, acc, gs0, gs1):
        cid = lax.axis_index("c")
        sid = lax.axis_index("s")
        wid = sid * NC + cid
        cbase = wid * nch
        ebase = wid * ept

        zero16 = jnp.zeros((L,), F32)
        dl = d // L

        pltpu.sync_copy(rc_hbm.at[cbase], idx0)
        pltpu.async_copy(x_hbm.at[idx0.at[1]], xb0, gs0)

        def zb(i, carry):
            zbuf[i // dl, pl.ds((i % dl) * L, L)] = zero16
            return carry
        lax.fori_loop(0, zrows * dl, zb, 0)

        def zc(k, carry):
            pltpu.sync_copy(zbuf, acc.at[pl.ds(sid * rpt + k * zrows, zrows)])
            return carry
        lax.fori_loop(0, nz, zc, 0)
        plsc.subcore_barrier()

        def process(c, idxc, idxn, xbc, xbn, gsc, gsn):
            @pl.when(c + 1 < nch)
            def _():
                pltpu.sync_copy(rc_hbm.at[cbase + c + 1], idxn)
                pltpu.async_copy(x_hbm.at[idxn.at[1]], xbn, gsn)

            @pl.when(c % cps == 0)
            def _():
                pltpu.sync_copy(
                    ex_hbm.at[pl.ds(ebase + (c // cps) * scg, scg)], exb)

            pltpu.make_async_copy(x_hbm.at[idxc.at[1]], xbc, gsc).wait()

            eoff = (c % cps) * cg

            def rb(g, carry2):
                w16 = exb[pl.ds(eoff + g * L, L)]
                for t in range(L):
                    i = g * L + t
                    w = w16[t]
                    for j in range(dl):
                        xbc[i, pl.ds(j * L, L)] = xbc[i, pl.ds(j * L, L)] * w
                return carry2
            lax.fori_loop(0, cg // L, rb, 0)

            pltpu.sync_copy(xbc, acc.at[idxc.at[0]], add=True)

        def chunk(c, carry):
            @pl.when(c % 2 == 0)
            def _():
                process(c, idx0, idx1, xb0, xb1, gs0, gs1)

            @pl.when(c % 2 == 1)
            def _():
                process(c, idx1, idx0, xb1, xb0, gs1, gs0)
            return carry
        lax.fori_loop(0, nch, chunk, 0)
        plsc.subcore_barrier()

        def ob(k, carry):
            off = sid * rpt + k * zrows
            pltpu.sync_copy(acc.at[pl.ds(off, zrows)], zbuf)
            pltpu.sync_copy(zbuf, opart_hbm.at[cid, pl.ds(off, zrows)])
            return carry
        lax.fori_loop(0, nz, ob, 0)

    return edge_b


def kernel(h, edge_index, W11, b11, W12, b12, W13, b13,
           W21, b21, W22, b22, W23, b23):
    n, d = h.shape
    e = edge_index.shape[1]
    blk = 2000
    cg = 80

    row = edge_index[0]
    col = edge_index[1]
    rc = jnp.stack([row.reshape(e // cg, cg), col.reshape(e // cg, cg)],
                   axis=1)

    edge_a = _make_edge_a(n, e)
    edge_b = _make_edge_b(n, e, d)

    def layer(s, x):
        ex = edge_a(s, row, col)
        return edge_b(x, rc, ex)

    s1, x1 = _tc_dense(h, W11, b11.reshape(1, d), W12.reshape(1, d),
                       W13, b13.reshape(1, d), blk)
    s1 = s1.reshape(n) + 0.5 * b12[0]
    opart1 = layer(s1, x1)

    s2, x2 = _tc_norm_dense(opart1, W21, b21.reshape(1, d),
                            W22.reshape(1, d), W23, b23.reshape(1, d), blk, n)
    s2 = s2.reshape(n) + 0.5 * b22[0]
    opart2 = layer(s2, x2)

    return _tc_final(opart2, blk, n)


# speedup vs baseline: 32.0441x; 1.1646x over previous
import functools
import jax
import jax.numpy as jnp
from jax import lax
from jax.experimental import pallas as pl
from jax.experimental.pallas import tpu as pltpu
from jax.experimental.pallas import tpu_sc as plsc

F32 = jnp.float32

NC = 2
NS = 16
NW = NC * NS
L = 16

_MESH = plsc.VectorSubcoreMesh(core_axis_name="c", subcore_axis_name="s")


def _dense_body(h_ref, Wa_ref, ba_ref, ws_ref, Wv_ref, bv_ref, s_ref, x_ref):
    hb = h_ref[...]
    e = jnp.tanh(jnp.dot(hb, Wa_ref[...], preferred_element_type=F32)
                 + ba_ref[...])
    s_ref[...] = jnp.sum(e * ws_ref[...], axis=1)[:, None]
    x_ref[...] = jnp.tanh(jnp.dot(hb, Wv_ref[...], preferred_element_type=F32)
                          + bv_ref[...])


def _norm_rows(o, d):
    mu = jnp.mean(o, axis=1, keepdims=True)
    var = jnp.sum((o - mu) * (o - mu), axis=1, keepdims=True) * (1.0 / (d - 1))
    return o / jnp.sqrt(var)


def _norm_dense_body(p_ref, Wa_ref, ba_ref, ws_ref, Wv_ref, bv_ref,
                     s_ref, x_ref):
    o = p_ref[0] + p_ref[1]
    hb = _norm_rows(o, o.shape[1])
    e = jnp.tanh(jnp.dot(hb, Wa_ref[...], preferred_element_type=F32)
                 + ba_ref[...])
    s_ref[...] = jnp.sum(e * ws_ref[...], axis=1)[:, None]
    x_ref[...] = jnp.tanh(jnp.dot(hb, Wv_ref[...], preferred_element_type=F32)
                          + bv_ref[...])


def _final_body(p_ref, out_ref):
    o = p_ref[0] + p_ref[1]
    out_ref[...] = _norm_rows(o, o.shape[1])


def _tc_dense(h, Wa, ba, ws, Wv, bv, blk):
    n, d = h.shape
    grid = (n // blk,)
    wspec = pl.BlockSpec((d, d), lambda i: (0, 0))
    bspec = pl.BlockSpec((1, d), lambda i: (0, 0))
    return pl.pallas_call(
        _dense_body,
        grid=grid,
        in_specs=[pl.BlockSpec((blk, d), lambda i: (i, 0)),
                  wspec, bspec, bspec, wspec, bspec],
        out_specs=[pl.BlockSpec((blk, 1), lambda i: (i, 0)),
                   pl.BlockSpec((blk, d), lambda i: (i, 0))],
        out_shape=[jax.ShapeDtypeStruct((n, 1), F32),
                   jax.ShapeDtypeStruct((n, d), F32)],
    )(h, Wa, ba, ws, Wv, bv)


def _tc_norm_dense(p, Wa, ba, ws, Wv, bv, blk, n):
    d = p.shape[2]
    grid = (n // blk,)
    wspec = pl.BlockSpec((d, d), lambda i: (0, 0))
    bspec = pl.BlockSpec((1, d), lambda i: (0, 0))
    return pl.pallas_call(
        _norm_dense_body,
        grid=grid,
        in_specs=[pl.BlockSpec((NC, blk, d), lambda i: (0, i, 0)),
                  wspec, bspec, bspec, wspec, bspec],
        out_specs=[pl.BlockSpec((blk, 1), lambda i: (i, 0)),
                   pl.BlockSpec((blk, d), lambda i: (i, 0))],
        out_shape=[jax.ShapeDtypeStruct((n, 1), F32),
                   jax.ShapeDtypeStruct((n, d), F32)],
    )(p, Wa, ba, ws, Wv, bv)


def _tc_final(p, blk, n):
    d = p.shape[2]
    grid = (n // blk,)
    return pl.pallas_call(
        _final_body,
        grid=grid,
        in_specs=[pl.BlockSpec((NC, blk, d), lambda i: (0, i, 0))],
        out_specs=pl.BlockSpec((blk, d), lambda i: (i, 0)),
        out_shape=jax.ShapeDtypeStruct((n, d), F32),
    )(p)


def _make_edge_a(n, e):
    ept = e // NW

    @functools.partial(
        pl.kernel,
        mesh=_MESH,
        out_type=jax.ShapeDtypeStruct((e,), F32),
        scratch_types=[pltpu.VMEM((n,), F32),
                       pltpu.VMEM((ept,), jnp.int32),
                       pltpu.VMEM((ept,), jnp.int32),
                       pltpu.VMEM((ept,), F32)],
        compiler_params=pltpu.CompilerParams(needs_layout_passes=False),
    )
    def edge_a(s_hbm, row_hbm, col_hbm, ex_hbm, s_v, row_v, col_v, ex_v):
        cid = lax.axis_index("c")
        sid = lax.axis_index("s")
        wid = sid * NC + cid
        base = wid * ept
        pltpu.sync_copy(s_hbm, s_v)
        pltpu.sync_copy(row_hbm.at[pl.ds(base, ept)], row_v)
        pltpu.sync_copy(col_hbm.at[pl.ds(base, ept)], col_v)

        def eb(j, carry):
            off = j * L
            r16 = row_v[pl.ds(off, L)]
            c16 = col_v[pl.ds(off, L)]
            sr = plsc.load_gather(s_v, [r16])
            sc_ = plsc.load_gather(s_v, [c16])
            z = sr + sc_
            u = jnp.exp(z + z)
            t = 1.0 - 2.0 / (u + 1.0)
            ex_v[pl.ds(off, L)] = jnp.exp(t)
            return carry
        lax.fori_loop(0, ept // L, eb, 0)

        pltpu.sync_copy(ex_v, ex_hbm.at[pl.ds(base, ept)])

    return edge_a


def _make_edge_b(n, e, d):
    ept = e // NW
    cg = 80
    nch = ept // cg
    scg = 2000
    cps = scg // cg
    zrows = 16
    rpt = -(-(n // NS) // zrows) * zrows
    nz = rpt // zrows
    npad = NS * rpt

    @functools.partial(
        pl.kernel,
        mesh=_MESH,
        out_type=jax.ShapeDtypeStruct((NC, npad, d), F32),
        scratch_types=[pltpu.VMEM((2, cg), jnp.int32),
                       pltpu.VMEM((2, cg), jnp.int32),
                       pltpu.VMEM((2, cg), jnp.int32),
                       pltpu.VMEM((scg,), F32),
                       pltpu.VMEM((cg, d), F32),
                       pltpu.VMEM((cg, d), F32),
                       pltpu.VMEM((cg, d), F32),
                       pltpu.VMEM((zrows, d), F32),
                       pltpu.VMEM_SHARED((npad, d), F32),
                       pltpu.SemaphoreType.DMA,
                       pltpu.SemaphoreType.DMA,
                       pltpu.SemaphoreType.DMA,
                       pltpu.SemaphoreType.DMA,
                       pltpu.SemaphoreType.DMA,
                       pltpu.SemaphoreType.DMA],
        compiler_params=pltpu.CompilerParams(needs_layout_passes=False),
    )
    def edge_b(x_hbm, rc_hbm, ex_hbm, opart_hbm,
               idx0, idx1, idx2, exb, xb0, xb1, xb2, zbuf, acc,
               gs0, gs1, gs2, ss0, ss1, ss2):
        cid = lax.axis_index("c")
        sid = lax.axis_index("s")
        wid = sid * NC + cid
        cbase = wid * nch
        ebase = wid * ept

        zero16 = jnp.zeros((L,), F32)
        dl = d // L
        idxs = (idx0, idx1, idx2)
        xbs = (xb0, xb1, xb2)
        gss = (gs0, gs1, gs2)
        sss = (ss0, ss1, ss2)

        pltpu.sync_copy(rc_hbm.at[cbase], idx0)
        pltpu.async_copy(x_hbm.at[idx0.at[1]], xb0, gs0)

        def zb(i, carry):
            zbuf[i // dl, pl.ds((i % dl) * L, L)] = zero16
            return carry
        lax.fori_loop(0, zrows * dl, zb, 0)

        def zc(k, carry):
            pltpu.sync_copy(zbuf, acc.at[pl.ds(sid * rpt + k * zrows, zrows)])
            return carry
        lax.fori_loop(0, nz, zc, 0)
        plsc.subcore_barrier()

        def process(c, idxc, idxn, xbc, xbn, gsc, gsn, ssc, ssn):
            @pl.when(c + 1 < nch)
            def _():
                @pl.when(c >= 2)
                def _():
                    pltpu.make_async_copy(
                        xbn, acc.at[idxn.at[0]], ssn).wait()
                pltpu.sync_copy(rc_hbm.at[cbase + c + 1], idxn)
                pltpu.async_copy(x_hbm.at[idxn.at[1]], xbn, gsn)

            @pl.when(c % cps == 0)
            def _():
                pltpu.sync_copy(
                    ex_hbm.at[pl.ds(ebase + (c // cps) * scg, scg)], exb)

            pltpu.make_async_copy(x_hbm.at[idxc.at[1]], xbc, gsc).wait()

            eoff = (c % cps) * cg

            def rb(g, carry2):
                w16 = exb[pl.ds(eoff + g * L, L)]
                for t in range(L):
                    i = g * L + t
                    w = w16[t]
                    for j in range(dl):
                        xbc[i, pl.ds(j * L, L)] = xbc[i, pl.ds(j * L, L)] * w
                return carry2
            lax.fori_loop(0, cg // L, rb, 0)

            pltpu.async_copy(xbc, acc.at[idxc.at[0]], ssc, add=True)

        def chunk(c, carry):
            for k in range(3):
                @pl.when(c % 3 == k)
                def _():
                    process(c, idxs[k], idxs[(k + 1) % 3],
                            xbs[k], xbs[(k + 1) % 3],
                            gss[k], gss[(k + 1) % 3],
                            sss[k], sss[(k + 1) % 3])
            return carry
        lax.fori_loop(0, nch, chunk, 0)

        for c in (nch - 3, nch - 2, nch - 1):
            k = c % 3
            pltpu.make_async_copy(xbs[k], acc.at[idxs[k].at[0]],
                                  sss[k]).wait()
        plsc.subcore_barrier()

        def ob(k, carry):
            off = sid * rpt + k * zrows
            pltpu.sync_copy(acc.at[pl.ds(off, zrows)], zbuf)
            pltpu.sync_copy(zbuf, opart_hbm.at[cid, pl.ds(off, zrows)])
            return carry
        lax.fori_loop(0, nz, ob, 0)

    return edge_b


def kernel(h, edge_index, W11, b11, W12, b12, W13, b13,
           W21, b21, W22, b22, W23, b23):
    n, d = h.shape
    e = edge_index.shape[1]
    blk = 2000
    cg = 80

    row = edge_index[0]
    col = edge_index[1]
    rc = jnp.stack([row.reshape(e // cg, cg), col.reshape(e // cg, cg)],
                   axis=1)

    edge_a = _make_edge_a(n, e)
    edge_b = _make_edge_b(n, e, d)

    def layer(s, x):
        ex = edge_a(s, row, col)
        return edge_b(x, rc, ex)

    s1, x1 = _tc_dense(h, W11, b11.reshape(1, d), W12.reshape(1, d),
                       W13, b13.reshape(1, d), blk)
    s1 = s1.reshape(n) + 0.5 * b12[0]
    opart1 = layer(s1, x1)

    s2, x2 = _tc_norm_dense(opart1, W21, b21.reshape(1, d),
                            W22.reshape(1, d), W23, b23.reshape(1, d), blk, n)
    s2 = s2.reshape(n) + 0.5 * b22[0]
    opart2 = layer(s2, x2)

    return _tc_final(opart2, blk, n)
